# baseline (speedup 1.0000x reference)
import jax
import jax.numpy as jnp
from jax import lax
from jax.experimental import pallas as pl
from jax.experimental.pallas import tpu as pltpu
from jax.experimental.pallas import tpu_sc as plsc

_NC = 2
_NS = 16
_NW = _NC * _NS
_CHUNK = 88
_ZROWS = 16


def _sc_msg_agg(ht_hbm, eae_hbm, idx_hbm, out_hbm,
                idx00, idx01, idx10, idx11,
                rows_g0, rows_e0, rows_g1, rows_e1,
                zbuf, agg_sh,
                sg0, se0, sg1, se1, si00, si01, si10, si11):
    c = lax.axis_index("c")
    s = lax.axis_index("s")
    wid = c * _NS + s
    cpt = idx_hbm.shape[0] // _NW
    rows_per_tile = agg_sh.shape[0] // _NS
    base = wid * cpt

    rows_g = (rows_g0, rows_g1)
    rows_e = (rows_e0, rows_e1)
    idx = ((idx00, idx01), (idx10, idx11))
    sg = (sg0, sg1)
    se = (se0, se1)
    si = ((si00, si01), (si10, si11))

    pltpu.sync_copy(idx_hbm.at[base + 0], idx[0][0])
    pltpu.async_copy(ht_hbm.at[idx[0][0].at[0]], rows_g[0], sg[0])
    pltpu.async_copy(eae_hbm.at[base + 0], rows_e[0], se[0])
    pltpu.sync_copy(idx_hbm.at[base + 1], idx[1][0])
    pltpu.async_copy(ht_hbm.at[idx[1][0].at[0]], rows_g[1], sg[1])
    pltpu.async_copy(eae_hbm.at[base + 1], rows_e[1], se[1])
    pltpu.async_copy(idx_hbm.at[base + 2], idx[0][1], si[0][1])
    pltpu.async_copy(idx_hbm.at[base + 3], idx[1][1], si[1][1])

    def _zb(i, carry):
        zbuf[i // 8, pl.ds((i % 8) * 16, 16)] = jnp.zeros((16,), jnp.float32)
        return carry
    lax.fori_loop(0, _ZROWS * 8, _zb, 0)
    for i in range(rows_per_tile // _ZROWS):
        pltpu.sync_copy(zbuf,
                        agg_sh.at[pl.ds(s * rows_per_tile + i * _ZROWS, _ZROWS)])
    plsc.subcore_barrier()

    def _visit(i, sl, q):
        pltpu.make_async_copy(eae_hbm.at[0], rows_g[sl], sg[sl]).wait()
        pltpu.make_async_copy(eae_hbm.at[0], rows_e[sl], se[sl]).wait()
        pltpu.sync_copy(rows_g[sl], agg_sh.at[idx[sl][q].at[1]], add=True)
        pltpu.sync_copy(rows_e[sl], agg_sh.at[idx[sl][q].at[1]], add=True)

        @pl.when(i + 4 < cpt)
        def _():
            pltpu.async_copy(idx_hbm.at[base + i + 4], idx[sl][q], si[sl][q])

        @pl.when(i + 2 < cpt)
        def _():
            pltpu.make_async_copy(idx_hbm.at[0], idx[sl][1 - q],
                                  si[sl][1 - q]).wait()
            pltpu.async_copy(ht_hbm.at[idx[sl][1 - q].at[0]], rows_g[sl],
                             sg[sl])
            pltpu.async_copy(eae_hbm.at[base + i + 2], rows_e[sl], se[sl])

    def _quad(t, carry):
        _visit(4 * t + 0, 0, 0)
        _visit(4 * t + 1, 1, 0)
        _visit(4 * t + 2, 0, 1)
        _visit(4 * t + 3, 1, 1)
        return carry
    lax.fori_loop(0, cpt // 4, _quad, 0)

    plsc.subcore_barrier()
    pltpu.sync_copy(agg_sh.at[pl.ds(s * rows_per_tile, rows_per_tile)],
                    out_hbm.at[c, pl.ds(s * rows_per_tile, rows_per_tile)])


def _mm(a_ref, w_ref, o_ref):
    o_ref[...] = jnp.dot(a_ref[...], w_ref[...],
                         preferred_element_type=jnp.float32)


def _tc_gates(mp_ref, h_ref, wz, uz, wr, ur, wn, un, out_ref):
    def dot(a, b):
        return jnp.dot(a, b, preferred_element_type=jnp.float32)
    m = mp_ref[0] + mp_ref[1]
    hb = h_ref[...]
    z = jax.nn.sigmoid(dot(m, wz[...]) + dot(hb, uz[...]))
    r = jax.nn.sigmoid(dot(m, wr[...]) + dot(hb, ur[...]))
    nt = jnp.tanh(dot(m, wn[...]) + dot(r * hb, un[...]))
    out_ref[...] = (1.0 - z) * hb + z * nt


def _rowmm_call(n_rows, k, d, blk):
    return pl.pallas_call(
        _mm,
        grid=(n_rows // blk,),
        in_specs=[pl.BlockSpec((blk, k), lambda i: (i, 0)),
                  pl.BlockSpec((k, d), lambda i: (0, 0))],
        out_specs=pl.BlockSpec((blk, d), lambda i: (i, 0)),
        out_shape=jax.ShapeDtypeStruct((n_rows, d), jnp.float32),
    )


def kernel(h, edge_index, edge_attr, Wmsg, We, Wz, Uz, Wr, Ur, Wn, Un):
    n, d = h.shape
    n_edges = edge_index.shape[1]
    de = edge_attr.shape[1]

    n_pad = ((n + _NS * _ZROWS - 1) // (_NS * _ZROWS)) * (_NS * _ZROWS)
    cpt = 4 * (-(-n_edges // (_NW * _CHUNK * 4)))
    e_pad = _NW * cpt * _CHUNK
    pad = e_pad - n_edges
    n_chunks = _NW * cpt
    src_f = jnp.concatenate([edge_index[0], jnp.zeros((pad,), jnp.int32)])
    dst_f = jnp.concatenate([edge_index[1], jnp.full((pad,), n, jnp.int32)])
    ea_p = jnp.concatenate([edge_attr, jnp.zeros((pad, de), jnp.float32)])
    idx_pack = jnp.stack([src_f.reshape(n_chunks, _CHUNK),
                          dst_f.reshape(n_chunks, _CHUNK)], axis=1)

    mesh = plsc.VectorSubcoreMesh(core_axis_name="c", subcore_axis_name="s",
                                  num_cores=_NC, num_subcores=_NS)
    msg_call = pl.kernel(
        _sc_msg_agg,
        out_type=jax.ShapeDtypeStruct((_NC, n_pad, d), jnp.float32),
        mesh=mesh,
        scratch_types=[
            pltpu.VMEM((2, _CHUNK), jnp.int32),
            pltpu.VMEM((2, _CHUNK), jnp.int32),
            pltpu.VMEM((2, _CHUNK), jnp.int32),
            pltpu.VMEM((2, _CHUNK), jnp.int32),
            pltpu.VMEM((_CHUNK, d), jnp.float32),
            pltpu.VMEM((_CHUNK, d), jnp.float32),
            pltpu.VMEM((_CHUNK, d), jnp.float32),
            pltpu.VMEM((_CHUNK, d), jnp.float32),
            pltpu.VMEM((_ZROWS, d), jnp.float32),
            pltpu.VMEM_SHARED((n_pad, d), jnp.float32),
        ] + [pltpu.SemaphoreType.DMA] * 8,
    )

    ht_call = _rowmm_call(n, d, d, 2000)
    eae_call = _rowmm_call(e_pad, de, d, _CHUNK * _NW)

    blk = 2000
    gate_call = pl.pallas_call(
        _tc_gates,
        grid=(n // blk,),
        in_specs=[
            pl.BlockSpec((_NC, blk, d), lambda i: (0, i, 0)),
            pl.BlockSpec((blk, d), lambda i: (i, 0)),
            pl.BlockSpec((d, d), lambda i: (0, 0)),
            pl.BlockSpec((d, d), lambda i: (0, 0)),
            pl.BlockSpec((d, d), lambda i: (0, 0)),
            pl.BlockSpec((d, d), lambda i: (0, 0)),
            pl.BlockSpec((d, d), lambda i: (0, 0)),
            pl.BlockSpec((d, d), lambda i: (0, 0)),
        ],
        out_specs=pl.BlockSpec((blk, d), lambda i: (i, 0)),
        out_shape=jax.ShapeDtypeStruct((n, d), jnp.float32),
    )

    num_layers = Wmsg.shape[0]
    for l in range(num_layers):
        ht = ht_call(h, Wmsg[l])
        eae = eae_call(ea_p, We[l]).reshape(n_chunks, _CHUNK, d)
        mp = msg_call(ht, eae, idx_pack)
        h = gate_call(mp, h, Wz[l], Uz[l], Wr[l], Ur[l], Wn[l], Un[l])
    return h


# speedup vs baseline: 1.5885x; 1.5885x over previous
import jax
import jax.numpy as jnp
from jax import lax
from jax.experimental import pallas as pl
from jax.experimental.pallas import tpu as pltpu
from jax.experimental.pallas import tpu_sc as plsc

_NC = 2
_NS = 16
_NW = _NC * _NS
_CHUNK = 88
_ZROWS = 16


def _sc_msg_agg(ht_hbm, eae_hbm, src_hbm, dst_hbm, out_hbm,
                src_c0, dst_c0, src_c1, dst_c1,
                rows_g0, rows_e0, rows_g1, rows_e1,
                zbuf, agg_sh, sg0, se0, sg1, se1):
    c = lax.axis_index("c")
    s = lax.axis_index("s")
    wid = c * _NS + s
    cpt = src_hbm.shape[0] // (_NW * _CHUNK)
    rows_per_tile = agg_sh.shape[0] // _NS
    base = wid * cpt

    def _load_idx(i, src_c, dst_c):
        pltpu.sync_copy(src_hbm.at[pl.ds((base + i) * _CHUNK, _CHUNK)], src_c)
        pltpu.sync_copy(dst_hbm.at[pl.ds((base + i) * _CHUNK, _CHUNK)], dst_c)

    _load_idx(0, src_c0, dst_c0)
    pltpu.async_copy(ht_hbm.at[src_c0], rows_g0, sg0)
    pltpu.async_copy(eae_hbm.at[base + 0], rows_e0, se0)
    _load_idx(1, src_c1, dst_c1)
    pltpu.async_copy(ht_hbm.at[src_c1], rows_g1, sg1)
    pltpu.async_copy(eae_hbm.at[base + 1], rows_e1, se1)

    def _zb(i, carry):
        zbuf[i // 8, pl.ds((i % 8) * 16, 16)] = jnp.zeros((16,), jnp.float32)
        return carry
    lax.fori_loop(0, _ZROWS * 8, _zb, 0)
    for i in range(rows_per_tile // _ZROWS):
        pltpu.sync_copy(zbuf,
                        agg_sh.at[pl.ds(s * rows_per_tile + i * _ZROWS, _ZROWS)])
    plsc.subcore_barrier()

    def _consume_prefetch(i, src_c, dst_c, rows_g, rows_e, sg, se):
        pltpu.make_async_copy(eae_hbm.at[0], rows_g, sg).wait()
        pltpu.make_async_copy(eae_hbm.at[0], rows_e, se).wait()
        pltpu.sync_copy(rows_g, agg_sh.at[dst_c], add=True)
        pltpu.sync_copy(rows_e, agg_sh.at[dst_c], add=True)

        @pl.when(i + 2 < cpt)
        def _():
            _load_idx(i + 2, src_c, dst_c)
            pltpu.async_copy(ht_hbm.at[src_c], rows_g, sg)
            pltpu.async_copy(eae_hbm.at[base + i + 2], rows_e, se)

    def _pair(p, carry):
        _consume_prefetch(2 * p, src_c0, dst_c0, rows_g0, rows_e0, sg0, se0)
        _consume_prefetch(2 * p + 1, src_c1, dst_c1, rows_g1, rows_e1, sg1, se1)
        return carry
    lax.fori_loop(0, cpt // 2, _pair, 0)

    plsc.subcore_barrier()
    pltpu.sync_copy(agg_sh.at[pl.ds(s * rows_per_tile, rows_per_tile)],
                    out_hbm.at[c, pl.ds(s * rows_per_tile, rows_per_tile)])


def _mm(a_ref, w_ref, o_ref):
    o_ref[...] = jnp.dot(a_ref[...], w_ref[...],
                         preferred_element_type=jnp.float32)


def _tc_gates(mp_ref, h_ref, wz, uz, wr, ur, wn, un, out_ref):
    def dot(a, b):
        return jnp.dot(a, b, preferred_element_type=jnp.float32)
    m = mp_ref[0] + mp_ref[1]
    hb = h_ref[...]
    z = jax.nn.sigmoid(dot(m, wz[...]) + dot(hb, uz[...]))
    r = jax.nn.sigmoid(dot(m, wr[...]) + dot(hb, ur[...]))
    nt = jnp.tanh(dot(m, wn[...]) + dot(r * hb, un[...]))
    out_ref[...] = (1.0 - z) * hb + z * nt


def _rowmm_call(n_rows, k, d, blk):
    return pl.pallas_call(
        _mm,
        grid=(n_rows // blk,),
        in_specs=[pl.BlockSpec((blk, k), lambda i: (i, 0)),
                  pl.BlockSpec((k, d), lambda i: (0, 0))],
        out_specs=pl.BlockSpec((blk, d), lambda i: (i, 0)),
        out_shape=jax.ShapeDtypeStruct((n_rows, d), jnp.float32),
    )


def kernel(h, edge_index, edge_attr, Wmsg, We, Wz, Uz, Wr, Ur, Wn, Un):
    n, d = h.shape
    n_edges = edge_index.shape[1]
    de = edge_attr.shape[1]

    n_pad = ((n + _NS * _ZROWS - 1) // (_NS * _ZROWS)) * (_NS * _ZROWS)
    cpt = 2 * (-(-n_edges // (_NW * _CHUNK * 2)))
    e_pad = _NW * cpt * _CHUNK
    pad = e_pad - n_edges
    n_chunks = _NW * cpt
    src_f = jnp.concatenate([edge_index[0], jnp.zeros((pad,), jnp.int32)])
    dst_f = jnp.concatenate([edge_index[1], jnp.full((pad,), n, jnp.int32)])
    ea_p = jnp.concatenate([edge_attr, jnp.zeros((pad, de), jnp.float32)])

    mesh = plsc.VectorSubcoreMesh(core_axis_name="c", subcore_axis_name="s",
                                  num_cores=_NC, num_subcores=_NS)
    msg_call = pl.kernel(
        _sc_msg_agg,
        out_type=jax.ShapeDtypeStruct((_NC, n_pad, d), jnp.float32),
        mesh=mesh,
        scratch_types=[
            pltpu.VMEM((_CHUNK,), jnp.int32),
            pltpu.VMEM((_CHUNK,), jnp.int32),
            pltpu.VMEM((_CHUNK,), jnp.int32),
            pltpu.VMEM((_CHUNK,), jnp.int32),
            pltpu.VMEM((_CHUNK, d), jnp.float32),
            pltpu.VMEM((_CHUNK, d), jnp.float32),
            pltpu.VMEM((_CHUNK, d), jnp.float32),
            pltpu.VMEM((_CHUNK, d), jnp.float32),
            pltpu.VMEM((_ZROWS, d), jnp.float32),
            pltpu.VMEM_SHARED((n_pad, d), jnp.float32),
            pltpu.SemaphoreType.DMA,
            pltpu.SemaphoreType.DMA,
            pltpu.SemaphoreType.DMA,
            pltpu.SemaphoreType.DMA,
        ],
    )

    ht_call = _rowmm_call(n, d, d, 2000)
    eae_call = _rowmm_call(e_pad, de, d, _CHUNK * _NW)

    blk = 2000
    gate_call = pl.pallas_call(
        _tc_gates,
        grid=(n // blk,),
        in_specs=[
            pl.BlockSpec((_NC, blk, d), lambda i: (0, i, 0)),
            pl.BlockSpec((blk, d), lambda i: (i, 0)),
            pl.BlockSpec((d, d), lambda i: (0, 0)),
            pl.BlockSpec((d, d), lambda i: (0, 0)),
            pl.BlockSpec((d, d), lambda i: (0, 0)),
            pl.BlockSpec((d, d), lambda i: (0, 0)),
            pl.BlockSpec((d, d), lambda i: (0, 0)),
            pl.BlockSpec((d, d), lambda i: (0, 0)),
        ],
        out_specs=pl.BlockSpec((blk, d), lambda i: (i, 0)),
        out_shape=jax.ShapeDtypeStruct((n, d), jnp.float32),
    )

    num_layers = Wmsg.shape[0]
    for l in range(num_layers):
        ht = ht_call(h, Wmsg[l])
        eae = eae_call(ea_p, We[l]).reshape(n_chunks, _CHUNK, d)
        mp = msg_call(ht, eae, src_f, dst_f)
        h = gate_call(mp, h, Wz[l], Uz[l], Wr[l], Ur[l], Wn[l], Un[l])
    return h
